# baseline (speedup 1.0000x reference)
import functools

import jax
import jax.numpy as jnp
from jax import lax
from jax.experimental import pallas as pl
from jax.experimental.pallas import tpu as pltpu
from jax.experimental.pallas import tpu_sc as plsc

SEQ_LEN = 200
BATCH = 4096
EMBED_DIM = 64
OUTPUT_SIZE = 128

NUM_CORES = 2
NUM_SUBCORES = 16
NUM_WORKERS = NUM_CORES * NUM_SUBCORES
B_PER_W = BATCH // NUM_WORKERS

NVEC = EMBED_DIM // 16


def _sc_pooled_sums(text, table):
    mesh = plsc.VectorSubcoreMesh(
        core_axis_name="c", subcore_axis_name="s",
        num_cores=NUM_CORES, num_subcores=NUM_SUBCORES)

    @functools.partial(
        pl.kernel,
        out_type=jax.ShapeDtypeStruct((BATCH, EMBED_DIM), jnp.float32),
        mesh=mesh,
        scratch_types=[
            pltpu.VMEM((SEQ_LEN, B_PER_W), jnp.int32),
            pltpu.VMEM((B_PER_W, EMBED_DIM), jnp.float32),
            pltpu.VMEM((B_PER_W, EMBED_DIM), jnp.float32),
            pltpu.VMEM((B_PER_W, EMBED_DIM), jnp.float32),
            pltpu.SemaphoreType.DMA,
            pltpu.SemaphoreType.DMA,
        ],
        compiler_params=pltpu.CompilerParams(use_tc_tiling_on_sc=False),
    )
    def sc_kernel(text_hbm, table_hbm, out_hbm, idx_v, buf0, buf1, sums_v,
                  sem0, sem1):
        wid = lax.axis_index("s") * NUM_CORES + lax.axis_index("c")
        base = wid * B_PER_W

        pltpu.sync_copy(text_hbm.at[:, pl.ds(base, B_PER_W)], idx_v)

        def zero_body(r, _):
            for j in range(NVEC):
                sums_v[r, pl.ds(16 * j, 16)] = jnp.zeros((16,), jnp.float32)
            return 0
        lax.fori_loop(0, B_PER_W, zero_body, 0)

        def accumulate(buf):
            def body(rr, _):
                for u in range(4):
                    r = rr * 4 + u
                    for j in range(NVEC):
                        plsc.addupdate(
                            sums_v.at[r, pl.ds(16 * j, 16)],
                            buf[r, pl.ds(16 * j, 16)])
                return 0
            lax.fori_loop(0, B_PER_W // 4, body, 0)

        pltpu.async_copy(table_hbm.at[idx_v.at[0]], buf0, sem0)

        def outer(i, _):
            l = 2 * i
            pltpu.async_copy(table_hbm.at[idx_v.at[l + 1]], buf1, sem1)
            pltpu.make_async_copy(table_hbm.at[idx_v.at[l]], buf0, sem0).wait()
            accumulate(buf0)

            @pl.when(l + 2 < SEQ_LEN)
            def _():
                pltpu.async_copy(table_hbm.at[idx_v.at[l + 2]], buf0, sem0)

            pltpu.make_async_copy(
                table_hbm.at[idx_v.at[l + 1]], buf1, sem1).wait()
            accumulate(buf1)
            return 0

        lax.fori_loop(0, SEQ_LEN // 2, outer, 0)

        pltpu.sync_copy(sums_v, out_hbm.at[pl.ds(base, B_PER_W)])

    return sc_kernel(text, table)


def _tc_linear(pooled_sums, W, b):
    def tc_kernel(x_ref, w_ref, b_ref, o_ref):
        x = x_ref[...]
        w = w_ref[...]
        acc = lax.dot_general(
            x, w, (((1,), (1,)), ((), ())),
            preferred_element_type=jnp.float32)
        o_ref[...] = acc * (1.0 / SEQ_LEN) + b_ref[...]

    return pl.pallas_call(
        tc_kernel,
        out_shape=jax.ShapeDtypeStruct((BATCH, OUTPUT_SIZE), jnp.float32),
        in_specs=[
            pl.BlockSpec(memory_space=pltpu.VMEM),
            pl.BlockSpec(memory_space=pltpu.VMEM),
            pl.BlockSpec(memory_space=pltpu.VMEM),
        ],
        out_specs=pl.BlockSpec(memory_space=pltpu.VMEM),
    )(pooled_sums, W, b.reshape(1, OUTPUT_SIZE))


def kernel(text, table, W, b):
    sums = _sc_pooled_sums(text.astype(jnp.int32), table)
    return _tc_linear(sums, W, b)


# speedup vs baseline: 2.5127x; 2.5127x over previous
import functools

import jax
import jax.numpy as jnp
from jax import lax
from jax.experimental import pallas as pl
from jax.experimental.pallas import tpu as pltpu
from jax.experimental.pallas import tpu_sc as plsc

SEQ_LEN = 200
BATCH = 4096
EMBED_DIM = 64
OUTPUT_SIZE = 128
VOCAB = 1000000

NUM_CORES = 2
NUM_SUBCORES = 16
NUM_WORKERS = NUM_CORES * NUM_SUBCORES
B_PER_W = BATCH // NUM_WORKERS

NVEC = EMBED_DIM // 16

RBLK = 32768
RHALF = RBLK // 2
RSHIFT = RHALF.bit_length() - 1
NBLK = (VOCAB + RBLK - 1) // RBLK
VPAD = NBLK * RBLK


def _tc_repack(table_t):
    def body(x_ref, o_ref):
        x = x_ref[...]
        xx = jnp.concatenate([x[:, 0:RHALF], x[:, RHALF:RBLK]], axis=0)
        o_ref[...] = jnp.transpose(xx)

    return pl.pallas_call(
        body,
        grid=(NBLK,),
        in_specs=[pl.BlockSpec((EMBED_DIM, RBLK), lambda g: (0, g))],
        out_specs=pl.BlockSpec((RHALF, 2 * EMBED_DIM), lambda g: (g, 0)),
        out_shape=jax.ShapeDtypeStruct((NBLK * RHALF, 2 * EMBED_DIM),
                                       jnp.float32),
    )(table_t)


def _sc_pooled_sums(text, table_view):
    mesh = plsc.VectorSubcoreMesh(
        core_axis_name="c", subcore_axis_name="s",
        num_cores=NUM_CORES, num_subcores=NUM_SUBCORES)

    @functools.partial(
        pl.kernel,
        out_type=jax.ShapeDtypeStruct((BATCH, EMBED_DIM), jnp.float32),
        mesh=mesh,
        scratch_types=[
            pltpu.VMEM((SEQ_LEN, B_PER_W), jnp.int32),
            pltpu.VMEM((B_PER_W, EMBED_DIM), jnp.float32),
            pltpu.VMEM((B_PER_W, EMBED_DIM), jnp.float32),
            pltpu.VMEM((B_PER_W, EMBED_DIM), jnp.float32),
            pltpu.VMEM((B_PER_W, EMBED_DIM), jnp.float32),
            pltpu.VMEM((B_PER_W, EMBED_DIM), jnp.float32),
            pltpu.VMEM((B_PER_W, EMBED_DIM), jnp.float32),
            pltpu.VMEM((B_PER_W, EMBED_DIM), jnp.float32),
            pltpu.VMEM((B_PER_W, EMBED_DIM), jnp.float32),
            pltpu.VMEM((B_PER_W, EMBED_DIM), jnp.float32),
        ] + [pltpu.SemaphoreType.DMA] * 8,
        compiler_params=pltpu.CompilerParams(use_tc_tiling_on_sc=False),
    )
    def sc_kernel(text_hbm, table_hbm, out_hbm, idx_v, buf0, buf1, buf2,
                  buf3, buf4, buf5, buf6, buf7, sums_v,
                  sem0, sem1, sem2, sem3, sem4, sem5, sem6, sem7):
        wid = lax.axis_index("s") * NUM_CORES + lax.axis_index("c")
        base = wid * B_PER_W

        pltpu.sync_copy(text_hbm.at[:, pl.ds(base, B_PER_W)], idx_v)

        def xform_body(l, _):
            for j in range(B_PER_W // 16):
                v = idx_v[l, pl.ds(16 * j, 16)]
                q = ((v & jnp.int32(-RBLK))
                     | ((v & jnp.int32(RHALF - 1)) << 1)
                     | ((v >> RSHIFT) & jnp.int32(1)))
                idx_v[l, pl.ds(16 * j, 16)] = q
            return 0
        lax.fori_loop(0, SEQ_LEN, xform_body, 0)

        def accumulate(buf):
            def body(rr, _):
                for u in range(8):
                    r = rr * 8 + u
                    for j in range(NVEC):
                        plsc.addupdate(
                            sums_v.at[r, pl.ds(16 * j, 16)],
                            buf[r, pl.ds(16 * j, 16)])
                return 0
            lax.fori_loop(0, B_PER_W // 8, body, 0)

        bufs = (buf0, buf1, buf2, buf3, buf4, buf5, buf6, buf7)
        sems = (sem0, sem1, sem2, sem3, sem4, sem5, sem6, sem7)
        NBUF = 8

        for b in range(NBUF):
            pltpu.async_copy(table_hbm.at[idx_v.at[b]], bufs[b], sems[b])

        def zero_body(r, _):
            for j in range(NVEC):
                sums_v[r, pl.ds(16 * j, 16)] = jnp.zeros((16,), jnp.float32)
            return 0
        lax.fori_loop(0, B_PER_W, zero_body, 0)

        def outer(i, _):
            l = i * NBUF
            for b in range(NBUF):
                tok = l + b
                pltpu.make_async_copy(
                    table_hbm.at[idx_v.at[tok]], bufs[b], sems[b]).wait()
                accumulate(bufs[b])

                @pl.when(tok + NBUF < SEQ_LEN)
                def _(b=b, tok=tok):
                    pltpu.async_copy(
                        table_hbm.at[idx_v.at[tok + NBUF]], bufs[b], sems[b])
            return 0

        lax.fori_loop(0, SEQ_LEN // NBUF, outer, 0)

        pltpu.sync_copy(sums_v, out_hbm.at[pl.ds(base, B_PER_W)])

    return sc_kernel(text, table_view)


def _tc_linear(pooled_sums, W, b):
    def tc_kernel(x_ref, w_ref, b_ref, o_ref):
        x = x_ref[...]
        w = w_ref[...]
        acc = lax.dot_general(
            x, w, (((1,), (1,)), ((), ())),
            preferred_element_type=jnp.float32)
        o_ref[...] = acc * (1.0 / SEQ_LEN) + b_ref[...]

    return pl.pallas_call(
        tc_kernel,
        out_shape=jax.ShapeDtypeStruct((BATCH, OUTPUT_SIZE), jnp.float32),
        in_specs=[
            pl.BlockSpec(memory_space=pltpu.VMEM),
            pl.BlockSpec(memory_space=pltpu.VMEM),
            pl.BlockSpec(memory_space=pltpu.VMEM),
        ],
        out_specs=pl.BlockSpec(memory_space=pltpu.VMEM),
    )(pooled_sums, W, b.reshape(1, OUTPUT_SIZE))


def kernel(text, table, W, b):
    table_t = jnp.transpose(table)
    packed = _tc_repack(table_t)
    packed = lax.optimization_barrier(packed)
    table_view = jnp.reshape(packed, (VPAD, EMBED_DIM))
    sums = _sc_pooled_sums(text.astype(jnp.int32), table_view)
    return _tc_linear(sums, W, b)
